# baseline (device time: 8683 ns/iter reference)
import jax
import jax.numpy as jnp
from jax import lax
from jax.experimental import pallas as pl
from jax.experimental.pallas import tpu as pltpu

N_DEV = 4
N_TOK = 256
D_IN = 128
D_OUT = 256
E_PER = 2
CAP = 25
TOK_PER = N_TOK // N_DEV


def kernel(x, router_W, route_idx, expert_W):
    def body(x_ref, rw_ref, idx_ref, w_ref, out_ref,
             pbuf, recv_buf, send_sems, recv_sems):
        my = lax.axis_index("i")

        barrier_sem = pltpu.get_barrier_semaphore()
        for off in range(1, N_DEV):
            pl.semaphore_signal(
                barrier_sem, inc=1,
                device_id=((my + off) % N_DEV,),
                device_id_type=pl.DeviceIdType.MESH,
            )
        pl.semaphore_wait(barrier_sem, N_DEV - 1)

        idx = idx_ref[:, :]
        col = lax.broadcasted_iota(jnp.int32, (N_TOK, E_PER), 1)
        local_e = E_PER * my + col
        onehot = (idx == local_e).astype(jnp.float32)
        ri = lax.broadcasted_iota(jnp.int32, (N_TOK, N_TOK), 0)
        ci = lax.broadcasted_iota(jnp.int32, (N_TOK, N_TOK), 1)
        tril = (ri >= ci).astype(jnp.float32)
        cum = jnp.dot(tril, onehot, preferred_element_type=jnp.float32)
        keep = onehot * (cum <= CAP).astype(jnp.float32)

        xb = x_ref[:, :].astype(jnp.bfloat16)
        acc = jnp.zeros((N_TOK, D_OUT), jnp.float32)
        for l in range(E_PER):
            gate = keep[:, l:l + 1].astype(jnp.bfloat16)
            w = w_ref[l, :, :].astype(jnp.bfloat16)
            acc = acc + jnp.dot(xb * gate, w, preferred_element_type=jnp.float32)
        pbuf[:, :] = acc.astype(jnp.bfloat16)

        rdmas = []
        for off in range(1, N_DEV):
            t = (my + off) % N_DEV
            rdma = pltpu.make_async_remote_copy(
                src_ref=pbuf.at[pl.ds(t * TOK_PER, TOK_PER), :],
                dst_ref=recv_buf.at[off - 1],
                send_sem=send_sems.at[off - 1],
                recv_sem=recv_sems.at[off - 1],
                device_id=(t,),
                device_id_type=pl.DeviceIdType.MESH,
            )
            rdma.start()
            rdmas.append(rdma)

        total = pbuf[pl.ds(my * TOK_PER, TOK_PER), :].astype(jnp.float32)
        for off in range(1, N_DEV):
            rdmas[off - 1].wait_recv()
            total = total + recv_buf[off - 1, :, :].astype(jnp.float32)
        out_ref[:, :] = total
        for rdma in rdmas:
            rdma.wait_send()

    return pl.pallas_call(
        body,
        out_shape=jax.ShapeDtypeStruct((TOK_PER, D_OUT), jnp.float32),
        in_specs=[
            pl.BlockSpec(memory_space=pltpu.VMEM),
            pl.BlockSpec(memory_space=pltpu.VMEM),
            pl.BlockSpec(memory_space=pltpu.VMEM),
            pl.BlockSpec(memory_space=pltpu.VMEM),
        ],
        out_specs=pl.BlockSpec(memory_space=pltpu.VMEM),
        scratch_shapes=[
            pltpu.VMEM((N_TOK, D_OUT), jnp.bfloat16),
            pltpu.VMEM((N_DEV - 1, TOK_PER, D_OUT), jnp.bfloat16),
            pltpu.SemaphoreType.DMA((N_DEV - 1,)),
            pltpu.SemaphoreType.DMA((N_DEV - 1,)),
        ],
        compiler_params=pltpu.CompilerParams(collective_id=0),
    )(x, router_W, route_idx, expert_W)
